# baseline (speedup 1.0000x reference)
import jax
import jax.numpy as jnp
from jax import lax
from jax.experimental import pallas as pl
from jax.experimental.pallas import tpu as pltpu
from jax.experimental.pallas import tpu_sc as plsc

B = 16384
D = 64
L = 16
NC = 2
NS = 16
NW = NC * NS
BPW = B // NW
N_ROWS = 1000000
ZROWS = N_ROWS // 2
BROWS = 7832
CHUNK = 64


def _mf_kernel(uid_hbm, iid_hbm, zu_hbm, zi_hbm, ub_hbm, ib_hbm, out_hbm,
               uid_v, iid_v, uzr_v, izr_v, ubr_v, ibr_v,
               urows_v, irows_v, ubrow_v, ibrow_v, out_v, sem):
    wid = lax.axis_index("s") * NC + lax.axis_index("c")
    base = wid * BPW
    pltpu.sync_copy(uid_hbm.at[pl.ds(base, BPW)], uid_v)
    pltpu.sync_copy(iid_hbm.at[pl.ds(base, BPW)], iid_v)

    zthr = jnp.full((L,), ZROWS, jnp.int32)

    @pl.loop(0, BPW, step=L)
    def _(i):
        u = uid_v[pl.ds(i, L)]
        t = iid_v[pl.ds(i, L)]
        uzr_v[pl.ds(i, L)] = jnp.where(u >= zthr, u - zthr, u)
        izr_v[pl.ds(i, L)] = jnp.where(t >= zthr, t - zthr, t)
        ubr_v[pl.ds(i, L)] = u >> 7
        ibr_v[pl.ds(i, L)] = t >> 7

    iota = lax.iota(jnp.int32, L)
    m127 = jnp.full((L,), 127, jnp.int32)
    dvec = jnp.full((L,), D, jnp.int32)
    zvec = jnp.zeros((L,), jnp.int32)

    for c in range(BPW // CHUNK):
        cb = c * CHUNK
        cps = [pltpu.async_copy(zu_hbm.at[uzr_v.at[pl.ds(cb, CHUNK)]],
                                urows_v, sem),
               pltpu.async_copy(zi_hbm.at[izr_v.at[pl.ds(cb, CHUNK)]],
                                irows_v, sem),
               pltpu.async_copy(ub_hbm.at[ubr_v.at[pl.ds(cb, CHUNK)]],
                                ubrow_v, sem),
               pltpu.async_copy(ib_hbm.at[ibr_v.at[pl.ds(cb, CHUNK)]],
                                ibrow_v, sem)]
        for cp in cps:
            cp.wait()

        @pl.loop(0, CHUNK, step=L)
        def _(rb):
            row_idx = rb + iota
            u = uid_v[pl.ds(cb + rb, L)]
            t = iid_v[pl.ds(cb + rb, L)]
            uoff = jnp.where(u >= zthr, dvec, zvec)
            ioff = jnp.where(t >= zthr, dvec, zvec)
            acc = (plsc.load_gather(ubrow_v, [row_idx, u & m127])
                   + plsc.load_gather(ibrow_v, [row_idx, t & m127]))
            for k in range(D):
                ck = jnp.full((L,), k, jnp.int32)
                acc = acc + (plsc.load_gather(urows_v, [row_idx, uoff + ck])
                             * plsc.load_gather(irows_v, [row_idx, ioff + ck]))
            out_v[pl.ds(cb + rb, L)] = acc

    pltpu.sync_copy(out_v, out_hbm.at[pl.ds(base, BPW)])


@jax.jit
def _mf(user_ids, item_ids, user_emb, item_emb, user_biases, item_biases):
    zu = jnp.concatenate([user_emb[:ZROWS], user_emb[ZROWS:]], axis=1)
    zi = jnp.concatenate([item_emb[:ZROWS], item_emb[ZROWS:]], axis=1)
    ub = jnp.pad(user_biases.reshape(-1),
                 (0, BROWS * 128 - N_ROWS)).reshape(BROWS, 128)
    ib = jnp.pad(item_biases.reshape(-1),
                 (0, BROWS * 128 - N_ROWS)).reshape(BROWS, 128)
    mesh = plsc.VectorSubcoreMesh(core_axis_name="c", subcore_axis_name="s")
    kfn = pl.kernel(
        _mf_kernel,
        mesh=mesh,
        compiler_params=pltpu.CompilerParams(needs_layout_passes=False),
        out_type=jax.ShapeDtypeStruct((B,), jnp.float32),
        scratch_types=[
            pltpu.VMEM((BPW,), jnp.int32),
            pltpu.VMEM((BPW,), jnp.int32),
            pltpu.VMEM((BPW,), jnp.int32),
            pltpu.VMEM((BPW,), jnp.int32),
            pltpu.VMEM((BPW,), jnp.int32),
            pltpu.VMEM((BPW,), jnp.int32),
            pltpu.VMEM((CHUNK, 2 * D), jnp.float32),
            pltpu.VMEM((CHUNK, 2 * D), jnp.float32),
            pltpu.VMEM((CHUNK, 128), jnp.float32),
            pltpu.VMEM((CHUNK, 128), jnp.float32),
            pltpu.VMEM((BPW,), jnp.float32),
            pltpu.SemaphoreType.DMA,
        ],
    )
    return kfn(user_ids, item_ids, zu, zi, ub, ib)


def kernel(user_ids, item_ids, user_emb, item_emb, user_biases, item_biases):
    return _mf(user_ids.astype(jnp.int32), item_ids.astype(jnp.int32),
               user_emb, item_emb, user_biases, item_biases)


# speedup vs baseline: 3.1332x; 3.1332x over previous
import jax
import jax.numpy as jnp
from jax import lax
from jax.experimental import pallas as pl
from jax.experimental.pallas import tpu as pltpu
from jax.experimental.pallas import tpu_sc as plsc

B = 16384
D = 64
L = 16
NC = 2
NS = 16
NW = NC * NS
BPW = B // NW
N_ROWS = 1000000
CBLK = 7813
GSTRIDE = CBLK * 8 * 128


def _mf_kernel(uid_hbm, iid_hbm, xu_hbm, xi_hbm, ub_hbm, ib_hbm, out_hbm,
               uid_v, iid_v, urb_v, irb_v, idx_v, urows_v, irows_v,
               ub_v, ib_v, out_v, sem):
    wid = lax.axis_index("s") * NC + lax.axis_index("c")
    base = wid * BPW
    pltpu.sync_copy(uid_hbm.at[pl.ds(base, BPW)], uid_v)
    pltpu.sync_copy(iid_hbm.at[pl.ds(base, BPW)], iid_v)

    m127 = jnp.full((L,), 127, jnp.int32)

    @pl.loop(0, BPW, step=L)
    def _(i):
        u = uid_v[pl.ds(i, L)]
        t = iid_v[pl.ds(i, L)]
        urb_v[pl.ds(i, L)] = ((u >> 7) << 10) + (u & m127)
        irb_v[pl.ds(i, L)] = ((t >> 7) << 10) + (t & m127)

    bias_cps = [pltpu.async_copy(ub_hbm.at[uid_v], ub_v, sem),
                pltpu.async_copy(ib_hbm.at[iid_v], ib_v, sem)]

    for rb_v, x_hbm, rows_v in ((urb_v, xu_hbm, urows_v),
                                (irb_v, xi_hbm, irows_v)):
        for e in range(D):
            ce = jnp.full((L,), (e >> 3) * GSTRIDE + (e & 7) * 128, jnp.int32)

            @pl.loop(0, BPW, step=L)
            def _(i):
                idx_v[e, pl.ds(i, L)] = rb_v[pl.ds(i, L)] + ce

        cps = [pltpu.async_copy(x_hbm.at[idx_v.at[e]], rows_v.at[e], sem)
               for e in range(D)]
        for cp in cps:
            cp.wait()

    for cp in bias_cps:
        cp.wait()

    @pl.loop(0, BPW, step=L)
    def _(i):
        acc = ub_v[pl.ds(i, L)] + ib_v[pl.ds(i, L)]
        for e in range(D):
            acc = acc + urows_v[e, pl.ds(i, L)] * irows_v[e, pl.ds(i, L)]
        out_v[pl.ds(i, L)] = acc

    pltpu.sync_copy(out_v, out_hbm.at[pl.ds(base, BPW)])


def _flat(table):
    zp = jnp.pad(table.T, ((0, 0), (0, CBLK * 128 - N_ROWS)))
    return zp.reshape(8, 8, CBLK, 128).transpose(0, 2, 1, 3).reshape(-1)


@jax.jit
def _mf(user_ids, item_ids, user_emb, item_emb, user_biases, item_biases):
    xu = _flat(user_emb)
    xi = _flat(item_emb)
    ub = user_biases.reshape(-1)
    ib = item_biases.reshape(-1)
    mesh = plsc.VectorSubcoreMesh(core_axis_name="c", subcore_axis_name="s")
    kfn = pl.kernel(
        _mf_kernel,
        mesh=mesh,
        compiler_params=pltpu.CompilerParams(
            needs_layout_passes=False, use_tc_tiling_on_sc=False),
        out_type=jax.ShapeDtypeStruct((B,), jnp.float32),
        scratch_types=[
            pltpu.VMEM((BPW,), jnp.int32),
            pltpu.VMEM((BPW,), jnp.int32),
            pltpu.VMEM((BPW,), jnp.int32),
            pltpu.VMEM((BPW,), jnp.int32),
            pltpu.VMEM((D, BPW), jnp.int32),
            pltpu.VMEM((D, BPW), jnp.float32),
            pltpu.VMEM((D, BPW), jnp.float32),
            pltpu.VMEM((BPW,), jnp.float32),
            pltpu.VMEM((BPW,), jnp.float32),
            pltpu.VMEM((BPW,), jnp.float32),
            pltpu.SemaphoreType.DMA,
        ],
    )
    return kfn(user_ids, item_ids, xu, xi, ub, ib)


def kernel(user_ids, item_ids, user_emb, item_emb, user_biases, item_biases):
    return _mf(user_ids.astype(jnp.int32), item_ids.astype(jnp.int32),
               user_emb, item_emb, user_biases, item_biases)


# speedup vs baseline: 3.2719x; 1.0443x over previous
import jax
import jax.numpy as jnp
from jax import lax
from jax.experimental import pallas as pl
from jax.experimental.pallas import tpu as pltpu
from jax.experimental.pallas import tpu_sc as plsc

B = 16384
D = 64
L = 16
NC = 2
NS = 16
NW = NC * NS
BPW = B // NW
N_ROWS = 1000000
CBLK = 7813
GSTRIDE = CBLK * 8 * 128

_M127 = None


def _flat_bases(ids_v, out_v):
    m127 = jnp.full((L,), 127, jnp.int32)

    @pl.loop(0, BPW, step=L)
    def _(i):
        u = ids_v[pl.ds(i, L)]
        out_v[pl.ds(i, L)] = ((u >> 7) << 10) + (u & m127)


def _gather_components(rb_v, x_hbm, idx_v, rows_v, sem):
    for e in range(D):
        ce = jnp.full((L,), (e >> 3) * GSTRIDE + (e & 7) * 128, jnp.int32)

        @pl.loop(0, BPW, step=L)
        def _(i):
            idx_v[e, pl.ds(i, L)] = rb_v[pl.ds(i, L)] + ce

    return [pltpu.async_copy(x_hbm.at[idx_v.at[e]], rows_v.at[e], sem)
            for e in range(D)]


def _k1(uid_hbm, xu_hbm, ub_hbm, gu_hbm, gub_hbm,
        uid_v, urb_v, idx_v, urows_v, ub_v, sem):
    wid = lax.axis_index("s") * NC + lax.axis_index("c")
    base = wid * BPW
    pltpu.sync_copy(uid_hbm.at[pl.ds(base, BPW)], uid_v)
    _flat_bases(uid_v, urb_v)
    cps = [pltpu.async_copy(ub_hbm.at[uid_v], ub_v, sem)]
    cps += _gather_components(urb_v, xu_hbm, idx_v, urows_v, sem)
    for cp in cps:
        cp.wait()
    pltpu.sync_copy(urows_v, gu_hbm.at[wid])
    pltpu.sync_copy(ub_v, gub_hbm.at[pl.ds(base, BPW)])


def _k2(iid_hbm, xi_hbm, ib_hbm, gu_hbm, gub_hbm, out_hbm,
        iid_v, irb_v, idx_v, irows_v, gu_v, gub_v, ib_v, out_v, sem):
    wid = lax.axis_index("s") * NC + lax.axis_index("c")
    base = wid * BPW
    pltpu.sync_copy(iid_hbm.at[pl.ds(base, BPW)], iid_v)
    _flat_bases(iid_v, irb_v)
    cps = [pltpu.async_copy(ib_hbm.at[iid_v], ib_v, sem)]
    cps += _gather_components(irb_v, xi_hbm, idx_v, irows_v, sem)
    pltpu.sync_copy(gu_hbm.at[wid], gu_v)
    pltpu.sync_copy(gub_hbm.at[pl.ds(base, BPW)], gub_v)
    for cp in cps:
        cp.wait()

    @pl.loop(0, BPW, step=L)
    def _(i):
        acc = gub_v[pl.ds(i, L)] + ib_v[pl.ds(i, L)]
        for e in range(D):
            acc = acc + gu_v[e, pl.ds(i, L)] * irows_v[e, pl.ds(i, L)]
        out_v[pl.ds(i, L)] = acc

    pltpu.sync_copy(out_v, out_hbm.at[pl.ds(base, BPW)])


def _flat(table):
    zp = jnp.pad(table.T, ((0, 0), (0, CBLK * 128 - N_ROWS)))
    return zp.reshape(8, 8, CBLK, 128).transpose(0, 2, 1, 3).reshape(-1)


@jax.jit
def _mf(user_ids, item_ids, user_emb, item_emb, user_biases, item_biases):
    xu = _flat(user_emb)
    xi = _flat(item_emb)
    ub = user_biases.reshape(-1)
    ib = item_biases.reshape(-1)
    mesh = plsc.VectorSubcoreMesh(core_axis_name="c", subcore_axis_name="s")
    cp = pltpu.CompilerParams(
        needs_layout_passes=False, use_tc_tiling_on_sc=False)
    k1 = pl.kernel(
        _k1,
        mesh=mesh,
        compiler_params=cp,
        out_type=(jax.ShapeDtypeStruct((NW, D, BPW), jnp.float32),
                  jax.ShapeDtypeStruct((B,), jnp.float32)),
        scratch_types=[
            pltpu.VMEM((BPW,), jnp.int32),
            pltpu.VMEM((BPW,), jnp.int32),
            pltpu.VMEM((D, BPW), jnp.int32),
            pltpu.VMEM((D, BPW), jnp.float32),
            pltpu.VMEM((BPW,), jnp.float32),
            pltpu.SemaphoreType.DMA,
        ],
    )
    gu, gub = k1(user_ids, xu, ub)
    k2 = pl.kernel(
        _k2,
        mesh=mesh,
        compiler_params=cp,
        out_type=jax.ShapeDtypeStruct((B,), jnp.float32),
        scratch_types=[
            pltpu.VMEM((BPW,), jnp.int32),
            pltpu.VMEM((BPW,), jnp.int32),
            pltpu.VMEM((D, BPW), jnp.int32),
            pltpu.VMEM((D, BPW), jnp.float32),
            pltpu.VMEM((D, BPW), jnp.float32),
            pltpu.VMEM((BPW,), jnp.float32),
            pltpu.VMEM((BPW,), jnp.float32),
            pltpu.VMEM((BPW,), jnp.float32),
            pltpu.SemaphoreType.DMA,
        ],
    )
    return k2(item_ids, xi, ib, gu, gub)


def kernel(user_ids, item_ids, user_emb, item_emb, user_biases, item_biases):
    return _mf(user_ids.astype(jnp.int32), item_ids.astype(jnp.int32),
               user_emb, item_emb, user_biases, item_biases)


# speedup vs baseline: 3.2735x; 1.0005x over previous
import jax
import jax.numpy as jnp
from jax import lax
from jax.experimental import pallas as pl
from jax.experimental.pallas import tpu as pltpu
from jax.experimental.pallas import tpu_sc as plsc

B = 16384
D = 64
L = 16
NC = 2
NS = 16
NW = NC * NS
BPW = B // NW
N_ROWS = 1000000
CBLK = 7813
GSTRIDE = CBLK * 8 * 128


def _flat_bases(ids_v, out_v):
    m127 = jnp.full((L,), 127, jnp.int32)

    @pl.loop(0, BPW, step=L)
    def _(i):
        u = ids_v[pl.ds(i, L)]
        out_v[pl.ds(i, L)] = ((u >> 7) << 10) + (u & m127)


def _gather_components(rb_v, x_hbm, idx_v, rows_v, sem):
    for e in range(D):
        ce = jnp.full((L,), (e >> 3) * GSTRIDE + (e & 7) * 128, jnp.int32)

        @pl.loop(0, BPW, step=L)
        def _(i):
            idx_v[e, pl.ds(i, L)] = rb_v[pl.ds(i, L)] + ce

    return [pltpu.async_copy(x_hbm.at[idx_v.at[e]], rows_v.at[e], sem)
            for e in range(D)]


def _k1(uid_hbm, xu_hbm, ub_hbm, gu_hbm, gub_hbm,
        uid_v, urb_v, idx_v, urows_v, ub_v, sem):
    wid = lax.axis_index("s") * NC + lax.axis_index("c")
    base = wid * BPW
    pltpu.sync_copy(uid_hbm.at[pl.ds(base, BPW)], uid_v)
    _flat_bases(uid_v, urb_v)
    cps = [pltpu.async_copy(ub_hbm.at[uid_v], ub_v, sem)]
    cps += _gather_components(urb_v, xu_hbm, idx_v, urows_v, sem)
    for cp in cps:
        cp.wait()
    pltpu.sync_copy(urows_v, gu_hbm.at[wid])
    pltpu.sync_copy(ub_v, gub_hbm.at[pl.ds(base, BPW)])


def _k2(iid_hbm, xi_hbm, ib_hbm, gu_hbm, gub_hbm, out_hbm,
        iid_v, irb_v, idx_v, irows_v, gu_v, gub_v, ib_v, out_v, sem):
    wid = lax.axis_index("s") * NC + lax.axis_index("c")
    base = wid * BPW
    pltpu.sync_copy(iid_hbm.at[pl.ds(base, BPW)], iid_v)
    _flat_bases(iid_v, irb_v)
    cps = [pltpu.async_copy(ib_hbm.at[iid_v], ib_v, sem)]
    cps += _gather_components(irb_v, xi_hbm, idx_v, irows_v, sem)
    pltpu.sync_copy(gu_hbm.at[wid], gu_v)
    pltpu.sync_copy(gub_hbm.at[pl.ds(base, BPW)], gub_v)
    for cp in cps:
        cp.wait()

    @pl.loop(0, BPW, step=L)
    def _(i):
        acc = gub_v[pl.ds(i, L)] + ib_v[pl.ds(i, L)]
        for e in range(D):
            acc = acc + gu_v[e, pl.ds(i, L)] * irows_v[e, pl.ds(i, L)]
        out_v[pl.ds(i, L)] = acc

    pltpu.sync_copy(out_v, out_hbm.at[pl.ds(base, BPW)])


def _flat(table):
    zp = jnp.pad(table.T, ((0, 0), (0, CBLK * 128 - N_ROWS)))
    return zp.reshape(8, 8, CBLK, 128).transpose(0, 2, 1, 3).reshape(-1)


@jax.jit
def _mf(user_ids, item_ids, user_emb, item_emb, user_biases, item_biases):
    xu = _flat(user_emb)
    xi = _flat(item_emb)
    ub = user_biases.reshape(-1)
    ib = item_biases.reshape(-1)
    mesh = plsc.VectorSubcoreMesh(core_axis_name="c", subcore_axis_name="s")
    cp = pltpu.CompilerParams(
        needs_layout_passes=False, use_tc_tiling_on_sc=False)
    k1 = pl.kernel(
        _k1,
        mesh=mesh,
        compiler_params=cp,
        out_type=(jax.ShapeDtypeStruct((NW, D, BPW), jnp.float32),
                  jax.ShapeDtypeStruct((B,), jnp.float32)),
        scratch_types=[
            pltpu.VMEM((BPW,), jnp.int32),
            pltpu.VMEM((BPW,), jnp.int32),
            pltpu.VMEM((D, BPW), jnp.int32),
            pltpu.VMEM((D, BPW), jnp.float32),
            pltpu.VMEM((BPW,), jnp.float32),
            pltpu.SemaphoreType.DMA,
        ],
    )
    gu, gub = k1(user_ids, xu, ub)
    k2 = pl.kernel(
        _k2,
        mesh=mesh,
        compiler_params=cp,
        out_type=jax.ShapeDtypeStruct((B,), jnp.float32),
        scratch_types=[
            pltpu.VMEM((BPW,), jnp.int32),
            pltpu.VMEM((BPW,), jnp.int32),
            pltpu.VMEM((D, BPW), jnp.int32),
            pltpu.VMEM((D, BPW), jnp.float32),
            pltpu.VMEM((D, BPW), jnp.float32),
            pltpu.VMEM((BPW,), jnp.float32),
            pltpu.VMEM((BPW,), jnp.float32),
            pltpu.VMEM((BPW,), jnp.float32),
            pltpu.SemaphoreType.DMA,
        ],
    )
    return k2(item_ids, xi, ib, gu, gub)


def kernel(user_ids, item_ids, user_emb, item_emb, user_biases, item_biases):
    return _mf(user_ids.astype(jnp.int32), item_ids.astype(jnp.int32),
               user_emb, item_emb, user_biases, item_biases)
